# baseline (speedup 1.0000x reference)
import jax
import jax.numpy as jnp
from jax import lax
from jax.experimental import pallas as pl
from jax.experimental.pallas import tpu as pltpu
from jax.experimental.pallas import tpu_sc as plsc

B = 4096
L = 200
D = 64
OUT = 32
NC = 2
NS = 16
NW = NC * NS
SPB = B // NW
LA = 104
LB = L - LA
NLANE = 16
ND = D // NLANE
NBUF = 6


def _pool_body(x_hbm, table_hbm, out_hbm, idx_v, rows_v, sums_v,
               maxs_v, *sems):
    wid = lax.axis_index("s") * NC + lax.axis_index("c")
    base = wid * SPB

    pltpu.sync_copy(x_hbm.at[pl.ds(base * L, SPB * L)], idx_v)

    def start_gather(s, buf):
        pltpu.async_copy(table_hbm.at[idx_v.at[pl.ds(s * L, LA)]],
                         rows_v.at[buf, pl.ds(0, LA)], sems[buf])
        pltpu.async_copy(table_hbm.at[idx_v.at[pl.ds(s * L + LA, LB)]],
                         rows_v.at[buf, pl.ds(LA, LB)], sems[buf])

    def wait_gather(buf):
        pltpu.make_async_copy(table_hbm.at[pl.ds(0, L)],
                              rows_v.at[buf], sems[buf]).wait()

    def compute(s, buf):
        def body(r, carry):
            sums = list(carry[:ND])
            maxs = list(carry[ND:])
            for d in range(ND):
                v = rows_v[buf, r, pl.ds(d * NLANE, NLANE)]
                sums[d] = sums[d] + v
                maxs[d] = jnp.maximum(maxs[d], v)
            return tuple(sums) + tuple(maxs)

        init = (tuple(jnp.zeros((NLANE,), jnp.float32) for _ in range(ND))
                + tuple(jnp.full((NLANE,), -jnp.inf, jnp.float32)
                        for _ in range(ND)))
        res = lax.fori_loop(0, L, body, init, unroll=8)
        for d in range(ND):
            sums_v[s, pl.ds(d * NLANE, NLANE)] = res[d]
            maxs_v[s, pl.ds(d * NLANE, NLANE)] = res[ND + d]

    for k in range(NBUF - 1):
        start_gather(k, k)

    def outer(i, carry):
        s0 = NBUF * i
        for k in range(NBUF):
            s = s0 + k

            @pl.when(s + NBUF - 1 < SPB)
            def _():
                start_gather(s + NBUF - 1, (k + NBUF - 1) % NBUF)

            wait_gather(k)
            compute(s, k)
        return carry

    lax.fori_loop(0, SPB // NBUF, outer, 0)

    for s in range(NBUF * (SPB // NBUF), SPB):
        wait_gather(s % NBUF)
        compute(s, s % NBUF)

    pltpu.sync_copy(sums_v, out_hbm.at[pl.ds(base, SPB)])
    pltpu.sync_copy(maxs_v, out_hbm.at[pl.ds(B + base, SPB)])


_pool = pl.kernel(
    _pool_body,
    out_type=jax.ShapeDtypeStruct((2 * B, D), jnp.float32),
    mesh=plsc.VectorSubcoreMesh(core_axis_name="c", subcore_axis_name="s"),
    compiler_params=pltpu.CompilerParams(use_tc_tiling_on_sc=False),
    scratch_types=[
        pltpu.VMEM((SPB * L,), jnp.int32),
        pltpu.VMEM((NBUF, L, D), jnp.float32),
        pltpu.VMEM((SPB, D), jnp.float32),
        pltpu.VMEM((SPB, D), jnp.float32),
    ] + [pltpu.SemaphoreType.DMA] * NBUF,
)


def _proj_body(reps_ref, lens_ref, w_ref, b_ref, out_ref):
    reps = reps_ref[...]
    means = reps[:B, :] / lens_ref[...]
    scaled = jnp.concatenate([means, reps[B:, :]], axis=0)
    out_ref[...] = lax.dot_general(
        scaled, w_ref[...], (((1,), (1,)), ((), ())),
        preferred_element_type=jnp.float32) + b_ref[...]


def kernel(x, lengths, emb_table, W, b):
    reps = _pool(x.astype(jnp.int32).reshape(B * L), emb_table)
    lens = lengths[1].astype(jnp.float32).reshape(B, 1)
    return pl.pallas_call(
        _proj_body,
        out_shape=jax.ShapeDtypeStruct((2 * B, OUT), jnp.float32),
    )(reps, lens, W, b.reshape(1, OUT))


# speedup vs baseline: 1.0172x; 1.0172x over previous
import jax
import jax.numpy as jnp
from jax import lax
from jax.experimental import pallas as pl
from jax.experimental.pallas import tpu as pltpu
from jax.experimental.pallas import tpu_sc as plsc

B = 4096
L = 200
D = 64
OUT = 32
NC = 2
NS = 16
NW = NC * NS
SPB = B // NW
LA = 104
LB = L - LA
NLANE = 16
ND = D // NLANE
NBUF = 4


def _pool_body(x_hbm, table_hbm, out_hbm, idx_v, rows_v, sums_v,
               maxs_v, *sems):
    wid = lax.axis_index("s") * NC + lax.axis_index("c")
    base = wid * SPB

    pltpu.sync_copy(x_hbm.at[pl.ds(base * L, SPB * L)], idx_v)

    def start_gather(s, buf):
        pltpu.async_copy(table_hbm.at[idx_v.at[pl.ds(s * L, LA)]],
                         rows_v.at[buf, pl.ds(0, LA)], sems[buf])
        pltpu.async_copy(table_hbm.at[idx_v.at[pl.ds(s * L + LA, LB)]],
                         rows_v.at[buf, pl.ds(LA, LB)], sems[buf])

    def wait_gather(buf):
        pltpu.make_async_copy(table_hbm.at[pl.ds(0, L)],
                              rows_v.at[buf], sems[buf]).wait()

    def compute(s, buf):
        def body(r, carry):
            sums = list(carry[:ND])
            maxs = list(carry[ND:])
            for d in range(ND):
                v = rows_v[buf, r, pl.ds(d * NLANE, NLANE)]
                sums[d] = sums[d] + v
                maxs[d] = jnp.maximum(maxs[d], v)
            return tuple(sums) + tuple(maxs)

        init = (tuple(jnp.zeros((NLANE,), jnp.float32) for _ in range(ND))
                + tuple(jnp.full((NLANE,), -jnp.inf, jnp.float32)
                        for _ in range(ND)))
        res = lax.fori_loop(0, L, body, init, unroll=8)
        for d in range(ND):
            sums_v[s, pl.ds(d * NLANE, NLANE)] = res[d]
            maxs_v[s, pl.ds(d * NLANE, NLANE)] = res[ND + d]

    for k in range(NBUF - 1):
        start_gather(k, k)

    def outer(i, carry):
        s0 = NBUF * i
        for k in range(NBUF):
            s = s0 + k

            @pl.when(s + NBUF - 1 < SPB)
            def _():
                start_gather(s + NBUF - 1, (k + NBUF - 1) % NBUF)

            wait_gather(k)
            compute(s, k)
        return carry

    lax.fori_loop(0, SPB // NBUF, outer, 0)

    for s in range(NBUF * (SPB // NBUF), SPB):
        wait_gather(s % NBUF)
        compute(s, s % NBUF)

    pltpu.sync_copy(sums_v, out_hbm.at[pl.ds(base, SPB)])
    pltpu.sync_copy(maxs_v, out_hbm.at[pl.ds(B + base, SPB)])


_pool = pl.kernel(
    _pool_body,
    out_type=jax.ShapeDtypeStruct((2 * B, D), jnp.float32),
    mesh=plsc.VectorSubcoreMesh(core_axis_name="c", subcore_axis_name="s"),
    compiler_params=pltpu.CompilerParams(use_tc_tiling_on_sc=False),
    scratch_types=[
        pltpu.VMEM((SPB * L,), jnp.int32),
        pltpu.VMEM((NBUF, L, D), jnp.float32),
        pltpu.VMEM((SPB, D), jnp.float32),
        pltpu.VMEM((SPB, D), jnp.float32),
    ] + [pltpu.SemaphoreType.DMA] * NBUF,
)


def _proj_body(reps_ref, lens_ref, w_ref, b_ref, out_ref):
    reps = reps_ref[...]
    means = reps[:B, :] / lens_ref[...]
    scaled = jnp.concatenate([means, reps[B:, :]], axis=0)
    out_ref[...] = lax.dot_general(
        scaled, w_ref[...], (((1,), (1,)), ((), ())),
        preferred_element_type=jnp.float32) + b_ref[...]


def kernel(x, lengths, emb_table, W, b):
    reps = _pool(x.astype(jnp.int32).reshape(B * L), emb_table)
    lens = lengths[1].astype(jnp.float32).reshape(B, 1)
    return pl.pallas_call(
        _proj_body,
        out_shape=jax.ShapeDtypeStruct((2 * B, OUT), jnp.float32),
    )(reps, lens, W, b.reshape(1, OUT))
